# baseline (speedup 1.0000x reference)
import jax
import jax.numpy as jnp
from jax.experimental import pallas as pl
from jax.experimental.pallas import tpu as pltpu


def _sda_body(h_ref, ctx_ref, w_in_ref, w_out_ref, out_ref, attn_ref,
              wctx_ref):
    tb, S, Dc = ctx_ref.shape
    h = h_ref[...]
    h16 = h.astype(jnp.bfloat16)

    target = jnp.dot(h, w_in_ref[...],
                     preferred_element_type=jnp.float32)
    target16 = target.astype(jnp.bfloat16)
    w16 = w_out_ref[...].astype(jnp.bfloat16)

    eye8 = jnp.eye(8, dtype=jnp.float32)
    nch = tb // 8

    for g in range(nch):
        c2b = (ctx_ref[g * 8:(g + 1) * 8, :, :]
               .reshape(8 * S, Dc).astype(jnp.bfloat16))
        r = jax.lax.dot_general(
            target16[g * 8:(g + 1) * 8, :], c2b,
            (((1,), (1,)), ((), ())),
            preferred_element_type=jnp.float32)
        attn_ref[g * 8:(g + 1) * 8, :] = jnp.sum(
            r.reshape(8, 8, S) * eye8[:, :, None], axis=0)

    logits = attn_ref[...]
    m = jnp.max(logits, axis=-1, keepdims=True)
    e = jnp.exp(logits - m)
    attn = e * (1.0 / jnp.sum(e, axis=-1, keepdims=True))
    attn_ref[...] = attn
    attn16 = attn.astype(jnp.bfloat16)

    eye16 = eye8.astype(jnp.bfloat16)
    for g in range(nch):
        c2b = (ctx_ref[g * 8:(g + 1) * 8, :, :]
               .reshape(8 * S, Dc).astype(jnp.bfloat16))
        ag = attn16[g * 8:(g + 1) * 8, :]
        a_bd = (ag[:, None, :] * eye16[:, :, None]).reshape(8, 8 * S)
        wctx_ref[g * 8:(g + 1) * 8, :] = jnp.dot(
            a_bd, c2b, preferred_element_type=jnp.float32)

    cat = jnp.concatenate(
        [wctx_ref[...].astype(jnp.bfloat16), h16], axis=-1)
    pre = jnp.dot(cat, w16, preferred_element_type=jnp.float32)
    out_ref[...] = jnp.tanh(pre).astype(out_ref.dtype)


def _pick_tile(B):
    for cand in (120, 320, 160, 128, 96, 240, 64, 32, 16, 8):
        if B % cand == 0 and (B // cand) % 2 == 0:
            return cand
    return B


def kernel(h, context, w_in, w_out_c, w_out_h):
    B, Dq = h.shape
    _, S, Dc = context.shape
    tile_b = _pick_tile(B)

    w_out = jnp.concatenate([w_out_c, w_out_h], axis=0)

    h_tilde, attn = pl.pallas_call(
        _sda_body,
        out_shape=(jax.ShapeDtypeStruct((B, Dq), h.dtype),
                   jax.ShapeDtypeStruct((B, S), jnp.float32)),
        grid=(B // tile_b,),
        in_specs=[
            pl.BlockSpec((tile_b, Dq), lambda i: (i, 0)),
            pl.BlockSpec((tile_b, S, Dc), lambda i: (i, 0, 0)),
            pl.BlockSpec((Dq, Dc), lambda i: (0, 0)),
            pl.BlockSpec((Dc + Dq, Dq), lambda i: (0, 0)),
        ],
        out_specs=(pl.BlockSpec((tile_b, Dq), lambda i: (i, 0)),
                   pl.BlockSpec((tile_b, S), lambda i: (i, 0))),
        scratch_shapes=[pltpu.VMEM((tile_b, Dc), jnp.float32)],
        compiler_params=pltpu.CompilerParams(
            dimension_semantics=("parallel",),
        ),
    )(h, context, w_in, w_out)
    return h_tilde, attn


# speedup vs baseline: 1.0460x; 1.0460x over previous
import jax
import jax.numpy as jnp
from jax.experimental import pallas as pl
from jax.experimental.pallas import tpu as pltpu


def _sda_body(h_ref, ctx_ref, w_in_ref, w_out_ref, out_ref, attn_ref,
              wctx_ref):
    tb, S, Dc = ctx_ref.shape
    h = h_ref[...]
    h16 = h.astype(jnp.bfloat16)

    target = jnp.dot(h, w_in_ref[...],
                     preferred_element_type=jnp.float32)
    target16 = target.astype(jnp.bfloat16)
    w16 = w_out_ref[...].astype(jnp.bfloat16)

    eye8 = jnp.eye(8, dtype=jnp.float32)
    nch = tb // 8

    for g in range(nch):
        c2b = (ctx_ref[g * 8:(g + 1) * 8, :, :]
               .reshape(8 * S, Dc).astype(jnp.bfloat16))
        r = jax.lax.dot_general(
            target16[g * 8:(g + 1) * 8, :], c2b,
            (((1,), (1,)), ((), ())),
            preferred_element_type=jnp.float32)
        attn_ref[g * 8:(g + 1) * 8, :] = jnp.sum(
            r.reshape(8, 8, S) * eye8[:, :, None], axis=0)

    logits = attn_ref[...]
    m = jnp.max(logits, axis=-1, keepdims=True)
    e = jnp.exp(logits - m)
    attn = e * (1.0 / jnp.sum(e, axis=-1, keepdims=True))
    attn_ref[...] = attn
    attn16 = attn.astype(jnp.bfloat16)

    eye16 = eye8.astype(jnp.bfloat16)
    for g in range(nch):
        c2b = (ctx_ref[g * 8:(g + 1) * 8, :, :]
               .reshape(8 * S, Dc).astype(jnp.bfloat16))
        ag = attn16[g * 8:(g + 1) * 8, :]
        a_bd = (ag[:, None, :] * eye16[:, :, None]).reshape(8, 8 * S)
        wctx_ref[g * 8:(g + 1) * 8, :] = jnp.dot(
            a_bd, c2b, preferred_element_type=jnp.float32)

    cat = jnp.concatenate(
        [wctx_ref[...].astype(jnp.bfloat16), h16], axis=-1)
    pre = jnp.dot(cat, w16, preferred_element_type=jnp.float32)
    out_ref[...] = jnp.tanh(pre).astype(out_ref.dtype)


def _pick_tile(B):
    for cand in (192, 320, 160, 128, 96, 240, 64, 32, 16, 8):
        if B % cand == 0 and (B // cand) % 2 == 0:
            return cand
    return B


def kernel(h, context, w_in, w_out_c, w_out_h):
    B, Dq = h.shape
    _, S, Dc = context.shape
    tile_b = _pick_tile(B)

    w_out = jnp.concatenate([w_out_c, w_out_h], axis=0)

    h_tilde, attn = pl.pallas_call(
        _sda_body,
        out_shape=(jax.ShapeDtypeStruct((B, Dq), h.dtype),
                   jax.ShapeDtypeStruct((B, S), jnp.float32)),
        grid=(B // tile_b,),
        in_specs=[
            pl.BlockSpec((tile_b, Dq), lambda i: (i, 0)),
            pl.BlockSpec((tile_b, S, Dc), lambda i: (i, 0, 0)),
            pl.BlockSpec((Dq, Dc), lambda i: (0, 0)),
            pl.BlockSpec((Dc + Dq, Dq), lambda i: (0, 0)),
        ],
        out_specs=(pl.BlockSpec((tile_b, Dq), lambda i: (i, 0)),
                   pl.BlockSpec((tile_b, S), lambda i: (i, 0))),
        scratch_shapes=[pltpu.VMEM((tile_b, Dc), jnp.float32)],
        compiler_params=pltpu.CompilerParams(
            dimension_semantics=("parallel",),
        ),
    )(h, context, w_in, w_out)
    return h_tilde, attn


# speedup vs baseline: 1.0604x; 1.0138x over previous
import jax
import jax.numpy as jnp
from jax.experimental import pallas as pl
from jax.experimental.pallas import tpu as pltpu


def _sda_body(h_ref, ctx_ref, w_in_ref, w_out_ref, out_ref, attn_ref,
              wctx_ref):
    tb, S, Dc = ctx_ref.shape
    h = h_ref[...]
    h16 = h.astype(jnp.bfloat16)

    target = jnp.dot(h, w_in_ref[...],
                     preferred_element_type=jnp.float32)
    target16 = target.astype(jnp.bfloat16)
    w16 = w_out_ref[...].astype(jnp.bfloat16)

    eye8 = jnp.eye(8, dtype=jnp.float32)
    nch = tb // 8

    for g in range(nch):
        c2b = (ctx_ref[g * 8:(g + 1) * 8, :, :]
               .reshape(8 * S, Dc).astype(jnp.bfloat16))
        r = jax.lax.dot_general(
            target16[g * 8:(g + 1) * 8, :], c2b,
            (((1,), (1,)), ((), ())),
            preferred_element_type=jnp.float32)
        attn_ref[g * 8:(g + 1) * 8, :] = jnp.sum(
            r.reshape(8, 8, S) * eye8[:, :, None], axis=0)

    logits = attn_ref[...]
    m = jnp.max(logits, axis=-1, keepdims=True)
    e = jnp.exp(logits - m)
    attn = e * (1.0 / jnp.sum(e, axis=-1, keepdims=True))
    attn_ref[...] = attn
    attn16 = attn.astype(jnp.bfloat16)

    eye16 = eye8.astype(jnp.bfloat16)
    for g in range(nch):
        c2b = (ctx_ref[g * 8:(g + 1) * 8, :, :]
               .reshape(8 * S, Dc).astype(jnp.bfloat16))
        ag = attn16[g * 8:(g + 1) * 8, :]
        a_bd = (ag[:, None, :] * eye16[:, :, None]).reshape(8, 8 * S)
        wctx_ref[g * 8:(g + 1) * 8, :] = jnp.dot(
            a_bd, c2b, preferred_element_type=jnp.float32)

    cat = jnp.concatenate(
        [wctx_ref[...].astype(jnp.bfloat16), h16], axis=-1)
    pre = jnp.dot(cat, w16, preferred_element_type=jnp.float32)
    out_ref[...] = jnp.tanh(pre).astype(out_ref.dtype)


def _pick_tile(B):
    for cand in (240, 320, 160, 128, 96, 192, 64, 32, 16, 8):
        if B % cand == 0 and (B // cand) % 2 == 0:
            return cand
    return B


def kernel(h, context, w_in, w_out_c, w_out_h):
    B, Dq = h.shape
    _, S, Dc = context.shape
    tile_b = _pick_tile(B)

    w_out = jnp.concatenate([w_out_c, w_out_h], axis=0)

    h_tilde, attn = pl.pallas_call(
        _sda_body,
        out_shape=(jax.ShapeDtypeStruct((B, Dq), h.dtype),
                   jax.ShapeDtypeStruct((B, S), jnp.float32)),
        grid=(B // tile_b,),
        in_specs=[
            pl.BlockSpec((tile_b, Dq), lambda i: (i, 0)),
            pl.BlockSpec((tile_b, S, Dc), lambda i: (i, 0, 0)),
            pl.BlockSpec((Dq, Dc), lambda i: (0, 0)),
            pl.BlockSpec((Dc + Dq, Dq), lambda i: (0, 0)),
        ],
        out_specs=(pl.BlockSpec((tile_b, Dq), lambda i: (i, 0)),
                   pl.BlockSpec((tile_b, S), lambda i: (i, 0))),
        scratch_shapes=[pltpu.VMEM((tile_b, Dc), jnp.float32)],
        compiler_params=pltpu.CompilerParams(
            dimension_semantics=("parallel",),
        ),
    )(h, context, w_in, w_out)
    return h_tilde, attn


# speedup vs baseline: 1.1218x; 1.0579x over previous
import jax
import jax.numpy as jnp
from jax.experimental import pallas as pl
from jax.experimental.pallas import tpu as pltpu


def _sda_body(h_ref, ctx_ref, w_in_ref, w_c_ref, w_h_ref, out_ref, attn_ref,
              wctx_ref):
    tb, S, Dc = ctx_ref.shape
    h = h_ref[...]
    h16 = h.astype(jnp.bfloat16)

    target = jnp.dot(h, w_in_ref[...],
                     preferred_element_type=jnp.float32)
    target16 = target.astype(jnp.bfloat16)

    eye8 = jnp.eye(8, dtype=jnp.float32)
    nch = tb // 8

    for g in range(nch):
        c2b = (ctx_ref[g * 8:(g + 1) * 8, :, :]
               .reshape(8 * S, Dc).astype(jnp.bfloat16))
        r = jax.lax.dot_general(
            target16[g * 8:(g + 1) * 8, :], c2b,
            (((1,), (1,)), ((), ())),
            preferred_element_type=jnp.float32)
        attn_ref[g * 8:(g + 1) * 8, :] = jnp.sum(
            r.reshape(8, 8, S) * eye8[:, :, None], axis=0)

    logits = attn_ref[...]
    m = jnp.max(logits, axis=-1, keepdims=True)
    e = jnp.exp(logits - m)
    attn = e * (1.0 / jnp.sum(e, axis=-1, keepdims=True))
    attn_ref[...] = attn
    attn16 = attn.astype(jnp.bfloat16)

    eye16 = eye8.astype(jnp.bfloat16)
    for g in range(nch):
        c2b = (ctx_ref[g * 8:(g + 1) * 8, :, :]
               .reshape(8 * S, Dc).astype(jnp.bfloat16))
        ag = attn16[g * 8:(g + 1) * 8, :]
        a_bd = (ag[:, None, :] * eye16[:, :, None]).reshape(8, 8 * S)
        wctx_ref[g * 8:(g + 1) * 8, :] = jnp.dot(
            a_bd, c2b, preferred_element_type=jnp.float32)

    pre = (jnp.dot(wctx_ref[...].astype(jnp.bfloat16),
                   w_c_ref[...].astype(jnp.bfloat16),
                   preferred_element_type=jnp.float32)
           + jnp.dot(h16, w_h_ref[...].astype(jnp.bfloat16),
                     preferred_element_type=jnp.float32))
    out_ref[...] = jnp.tanh(pre).astype(out_ref.dtype)


def _pick_tile(B):
    for cand in (240, 320, 160, 128, 96, 192, 64, 32, 16, 8):
        if B % cand == 0 and (B // cand) % 2 == 0:
            return cand
    return B


def kernel(h, context, w_in, w_out_c, w_out_h):
    B, Dq = h.shape
    _, S, Dc = context.shape
    tile_b = _pick_tile(B)

    h_tilde, attn = pl.pallas_call(
        _sda_body,
        out_shape=(jax.ShapeDtypeStruct((B, Dq), h.dtype),
                   jax.ShapeDtypeStruct((B, S), jnp.float32)),
        grid=(B // tile_b,),
        in_specs=[
            pl.BlockSpec((tile_b, Dq), lambda i: (i, 0)),
            pl.BlockSpec((tile_b, S, Dc), lambda i: (i, 0, 0)),
            pl.BlockSpec((Dq, Dc), lambda i: (0, 0)),
            pl.BlockSpec((Dc, Dq), lambda i: (0, 0)),
            pl.BlockSpec((Dq, Dq), lambda i: (0, 0)),
        ],
        out_specs=(pl.BlockSpec((tile_b, Dq), lambda i: (i, 0)),
                   pl.BlockSpec((tile_b, S), lambda i: (i, 0))),
        scratch_shapes=[pltpu.VMEM((tile_b, Dc), jnp.float32)],
        compiler_params=pltpu.CompilerParams(
            dimension_semantics=("parallel",),
        ),
    )(h, context, w_in, w_out_c, w_out_h)
    return h_tilde, attn
